# baseline (speedup 1.0000x reference)
import functools

import jax
import jax.numpy as jnp
from jax.experimental import pallas as pl
from jax.experimental.pallas import tpu as pltpu

_K = 64
_NCLUS = 8
_TEMP = 0.1
_BN_EPS = 1e-5
_TQ = 256


def _proj_kernel(f_ref, wc_ref, gamma_ref, beta_ref, mtail_ref,
                 feats_ref, qn_ref, mn_ref):
    x = jax.lax.dot_general(f_ref[...], wc_ref[...], (((1,), (1,)), ((), ())),
                            preferred_element_type=jnp.float32)
    mean = jnp.mean(x, axis=0, keepdims=True)
    var = jnp.mean((x - mean) ** 2, axis=0, keepdims=True)
    xn = (x - mean) / jnp.sqrt(var + _BN_EPS)
    feats = jnp.maximum(xn * gamma_ref[...] + beta_ref[...], 0.0)
    feats_ref[...] = feats

    qn = jnp.sum(feats * feats, axis=1, keepdims=True)
    qn_ref[...] = qn

    ones = jnp.ones((1, feats.shape[1]), jnp.float32)
    mn_q = jax.lax.dot_general(ones, feats * feats, (((1,), (1,)), ((), ())),
                               preferred_element_type=jnp.float32)
    mtail = mtail_ref[...]
    mn_t = jax.lax.dot_general(ones, mtail * mtail, (((1,), (1,)), ((), ())),
                               preferred_element_type=jnp.float32)
    mn_ref[...] = jnp.concatenate([mn_q, mn_t], axis=1)


def _density_kernel(q_ref, qn_ref, m_ref, mn_ref, dens_ref):
    prod = jax.lax.dot_general(q_ref[...], m_ref[...], (((1,), (1,)), ((), ())),
                               preferred_element_type=jnp.float32)
    d2 = jnp.maximum(qn_ref[...] + mn_ref[...] - 2.0 * prod, 1e-12)
    tq, mm = d2.shape

    col = jax.lax.broadcasted_iota(jnp.int32, (tq, mm), 1)
    row = jax.lax.broadcasted_iota(jnp.int32, (tq, mm), 0)
    self_col = row + pl.program_id(0) * tq
    d2_ns = jnp.where(col == self_col, jnp.float32(jnp.inf), d2)
    lo = jax.lax.bitcast_convert_type(
        jnp.min(d2_ns, axis=1, keepdims=True), jnp.int32)

    gmin = jnp.min(d2.reshape(tq, 128, mm // 128), axis=2)
    glo = jax.lax.bitcast_convert_type(
        jnp.min(gmin, axis=1, keepdims=True), jnp.int32)
    ghi = jax.lax.bitcast_convert_type(
        jnp.max(gmin, axis=1, keepdims=True), jnp.int32)

    def gbody(_, carry):
        glo, ghi = carry
        mid = glo + ((ghi - glo) >> 1)
        t = jax.lax.bitcast_convert_type(mid, jnp.float32)
        cnt = jnp.sum(jnp.where(gmin <= t, 1.0, 0.0), axis=1, keepdims=True)
        ge = cnt >= float(_K)
        return jnp.where(ge, glo, mid + 1), jnp.where(ge, mid, ghi)

    _, hi = jax.lax.fori_loop(0, 16, gbody, (glo, ghi))
    hi = jnp.maximum(hi, lo)

    def cond(carry):
        lo, hi = carry
        return jnp.any(lo < hi)

    def body(carry):
        lo, hi = carry
        mid = lo + ((hi - lo) >> 1)
        t = jax.lax.bitcast_convert_type(mid, jnp.float32)
        cnt = jnp.sum(jnp.where(d2 <= t, 1.0, 0.0), axis=1, keepdims=True)
        ge = cnt >= float(_K)
        return jnp.where(ge, lo, mid + 1), jnp.where(ge, mid, hi)

    lo, hi = jax.lax.while_loop(cond, body, (lo, hi))
    r2k = jax.lax.bitcast_convert_type(hi, jnp.float32)

    inv_bw2 = 1.0 / jnp.maximum(r2k, 1e-16)
    dens_ref[...] = jnp.sum(jnp.exp(-d2 * inv_bw2), axis=1, keepdims=True)


def _peaks_kernel(feats_ref, qn_ref, dens_ref, out_ref):
    nb = dens_ref.shape[0]
    hw = dens_ref.shape[1]
    d3 = dens_ref[nb - 1:nb, :]
    iota = jax.lax.broadcasted_iota(jnp.int32, (1, hw), 1)

    vals = d3
    top_v = []
    centers = []
    for _ in range(_NCLUS):
        m = jnp.max(vals)
        idx = jnp.min(jnp.where(vals == m, iota, jnp.int32(2 ** 30)))
        top_v.append(jnp.reshape(m, (1, 1)))
        centers.append(feats_ref[pl.ds((nb - 1) * hw + idx, 1), :])
        vals = jnp.where(iota == idx, -jnp.inf, vals)

    tv = jnp.concatenate(top_v, axis=1)
    cen = jnp.concatenate(centers, axis=0)
    priors = tv / (jnp.sum(tv) + 1e-8)

    ones = jnp.ones((1, cen.shape[1]), jnp.float32)
    cn = jax.lax.dot_general(ones, cen * cen, (((1,), (1,)), ((), ())),
                             preferred_element_type=jnp.float32)
    prod = jax.lax.dot_general(feats_ref[...], cen, (((1,), (1,)), ((), ())),
                               preferred_element_type=jnp.float32)
    d2 = jnp.maximum(qn_ref[...] + cn - 2.0 * prod, 1e-12)
    dist = jnp.sqrt(d2)
    logits = -dist / _TEMP
    logits = logits - jnp.max(logits, axis=1, keepdims=True)
    e = jnp.exp(logits)
    soft = e / jnp.sum(e, axis=1, keepdims=True)
    out_ref[...] = jnp.sum(soft * priors, axis=1, keepdims=True)


def kernel(features, Wc, gamma, beta, memory_bank, perm):
    del perm
    b, c, h, w = features.shape
    n = b * h * w
    m_total = memory_bank.shape[0]
    n_tail = m_total - n

    flat = features.reshape(b, c, h * w).transpose(0, 2, 1).reshape(n, c)
    mtail = memory_bank[n:]

    feats, qn, mn = pl.pallas_call(
        _proj_kernel,
        out_shape=(
            jax.ShapeDtypeStruct((n, c), jnp.float32),
            jax.ShapeDtypeStruct((n, 1), jnp.float32),
            jax.ShapeDtypeStruct((1, m_total), jnp.float32),
        ),
    )(flat, Wc, gamma.reshape(1, c), beta.reshape(1, c), mtail)

    m_all = jnp.concatenate([feats, mtail], axis=0)

    grid = n // _TQ
    dens = pl.pallas_call(
        _density_kernel,
        grid=(grid,),
        in_specs=[
            pl.BlockSpec((_TQ, c), lambda i: (i, 0)),
            pl.BlockSpec((_TQ, 1), lambda i: (i, 0)),
            pl.BlockSpec((m_total, c), lambda i: (0, 0)),
            pl.BlockSpec((1, m_total), lambda i: (0, 0)),
        ],
        out_specs=pl.BlockSpec((_TQ, 1), lambda i: (i, 0)),
        out_shape=jax.ShapeDtypeStruct((n, 1), jnp.float32),
    )(feats, qn, m_all, mn)

    sem = pl.pallas_call(
        _peaks_kernel,
        out_shape=jax.ShapeDtypeStruct((n, 1), jnp.float32),
    )(feats, qn, dens.reshape(b, h * w))

    return sem.reshape(b, 1, h, w)


# speedup vs baseline: 1.0064x; 1.0064x over previous
import functools

import jax
import jax.numpy as jnp
from jax import lax
from jax.experimental import pallas as pl
from jax.experimental.pallas import tpu as pltpu
from jax.experimental.pallas import tpu_sc as plsc

_K = 64
_NCLUS = 8
_TEMP = 0.1
_BN_EPS = 1e-5
_TQ = 256


def _proj_kernel(f_ref, wc_ref, gamma_ref, beta_ref, mtail_ref,
                 feats_ref, qn_ref, mn_ref):
    x = jax.lax.dot_general(f_ref[...], wc_ref[...], (((1,), (1,)), ((), ())),
                            preferred_element_type=jnp.float32)
    mean = jnp.mean(x, axis=0, keepdims=True)
    var = jnp.mean((x - mean) ** 2, axis=0, keepdims=True)
    xn = (x - mean) / jnp.sqrt(var + _BN_EPS)
    feats = jnp.maximum(xn * gamma_ref[...] + beta_ref[...], 0.0)
    feats_ref[...] = feats

    qn = jnp.sum(feats * feats, axis=1, keepdims=True)
    qn_ref[...] = qn

    ones = jnp.ones((1, feats.shape[1]), jnp.float32)
    mn_q = jax.lax.dot_general(ones, feats * feats, (((1,), (1,)), ((), ())),
                               preferred_element_type=jnp.float32)
    mtail = mtail_ref[...]
    mn_t = jax.lax.dot_general(ones, mtail * mtail, (((1,), (1,)), ((), ())),
                               preferred_element_type=jnp.float32)
    mn_ref[...] = jnp.concatenate([mn_q, mn_t], axis=1)


def _density_kernel(q_ref, qn_ref, m_ref, mn_ref, dens_ref):
    prod = jax.lax.dot_general(q_ref[...], m_ref[...], (((1,), (1,)), ((), ())),
                               preferred_element_type=jnp.float32)
    d2 = jnp.maximum(qn_ref[...] + mn_ref[...] - 2.0 * prod, 1e-12)
    tq, mm = d2.shape

    col = jax.lax.broadcasted_iota(jnp.int32, (tq, mm), 1)
    row = jax.lax.broadcasted_iota(jnp.int32, (tq, mm), 0)
    self_col = row + pl.program_id(0) * tq
    d2_ns = jnp.where(col == self_col, jnp.float32(jnp.inf), d2)
    lo = jax.lax.bitcast_convert_type(
        jnp.min(d2_ns, axis=1, keepdims=True), jnp.int32)

    gmin = jnp.min(d2.reshape(tq, 128, mm // 128), axis=2)
    glo = jax.lax.bitcast_convert_type(
        jnp.min(gmin, axis=1, keepdims=True), jnp.int32)
    ghi = jax.lax.bitcast_convert_type(
        jnp.max(gmin, axis=1, keepdims=True), jnp.int32)

    def gbody(_, carry):
        glo, ghi = carry
        mid = glo + ((ghi - glo) >> 1)
        t = jax.lax.bitcast_convert_type(mid, jnp.float32)
        cnt = jnp.sum(jnp.where(gmin <= t, 1.0, 0.0), axis=1, keepdims=True)
        ge = cnt >= float(_K)
        return jnp.where(ge, glo, mid + 1), jnp.where(ge, mid, ghi)

    _, hi = jax.lax.fori_loop(0, 16, gbody, (glo, ghi))
    hi = jnp.maximum(hi, lo)

    span0 = jnp.max(hi - lo)

    def cond(carry):
        _, _, span = carry
        return span > 0

    def body(carry):
        lo, hi, span = carry
        mid = lo + ((hi - lo) >> 1)
        t = jax.lax.bitcast_convert_type(mid, jnp.float32)
        cnt = jnp.sum(jnp.where(d2 <= t, 1.0, 0.0), axis=1, keepdims=True)
        ge = cnt >= float(_K)
        return (jnp.where(ge, lo, mid + 1), jnp.where(ge, mid, hi),
                span >> 1)

    lo, hi, _ = jax.lax.while_loop(cond, body, (lo, hi, span0))
    r2k = jax.lax.bitcast_convert_type(hi, jnp.float32)

    inv_bw2 = 1.0 / jnp.maximum(r2k, 1e-16)
    dens_ref[...] = jnp.sum(jnp.exp(-d2 * inv_bw2), axis=1, keepdims=True)


def _sc_peaks_body(dens_hbm, feats_hbm, cen_hbm, tv_hbm,
                   dens_v, cen_v, tv_v, idx_v, red_v, redi_v, sem, *, base):
    wid = lax.axis_index("s") * 2 + lax.axis_index("c")

    @pl.when(wid == 0)
    def _():
        pltpu.sync_copy(dens_hbm, dens_v)
        hw = dens_hbm.shape[0]
        nv = hw // 16
        lane = lax.iota(jnp.int32, 16)

        def splat_max_f32(x):
            for s in (8, 4, 2, 1):
                red_v[pl.ds(0, 16)] = x
                x = jnp.maximum(x, plsc.load_gather(red_v, [lane ^ s]))
            return x

        def splat_min_i32(x):
            for s in (8, 4, 2, 1):
                redi_v[pl.ds(0, 16)] = x
                x = jnp.minimum(x, plsc.load_gather(redi_v, [lane ^ s]))
            return x

        def round_body(r, carry):
            tv_vec, idx_vec = carry

            def mx_body(v, best):
                return jnp.maximum(best, dens_v[pl.ds(v * 16, 16)])
            best = lax.fori_loop(0, nv, mx_body,
                                 jnp.full((16,), -jnp.inf, jnp.float32))
            m = splat_max_f32(best)

            def ix_body(v, besti):
                x = dens_v[pl.ds(v * 16, 16)]
                gi = lane + v * 16
                return jnp.minimum(besti, jnp.where(x == m, gi,
                                                    jnp.int32(2 ** 30)))
            besti = lax.fori_loop(0, nv, ix_body,
                                  jnp.full((16,), 2 ** 30, jnp.int32))
            idx = splat_min_i32(besti)

            tv_vec = jnp.where(lane == r, m, tv_vec)
            idx_vec = jnp.where(lane == r, idx, idx_vec)
            plsc.store_scatter(dens_v, [idx],
                               jnp.full((16,), -jnp.inf, jnp.float32),
                               mask=lane == 0)
            return tv_vec, idx_vec

        tv_vec, idx_vec = lax.fori_loop(
            0, _NCLUS, round_body,
            (jnp.zeros((16,), jnp.float32), jnp.zeros((16,), jnp.int32)))

        tv_v[...] = tv_vec
        idx_v[...] = idx_vec + base

        pltpu.async_copy(feats_hbm.at[idx_v], cen_v, sem).wait()
        pltpu.sync_copy(cen_v, cen_hbm)
        pltpu.sync_copy(tv_v, tv_hbm)


def _sc_peaks(dens3, feats, base):
    c = feats.shape[1]
    hw = dens3.shape[0]
    mesh = plsc.VectorSubcoreMesh(core_axis_name="c", subcore_axis_name="s")
    fn = pl.kernel(
        functools.partial(_sc_peaks_body, base=base),
        mesh=mesh,
        compiler_params=pltpu.CompilerParams(needs_layout_passes=False),
        out_type=(
            jax.ShapeDtypeStruct((16, c), jnp.float32),
            jax.ShapeDtypeStruct((16,), jnp.float32),
        ),
        scratch_types=[
            pltpu.VMEM((hw,), jnp.float32),
            pltpu.VMEM((16, c), jnp.float32),
            pltpu.VMEM((16,), jnp.float32),
            pltpu.VMEM((16,), jnp.int32),
            pltpu.VMEM((128,), jnp.float32),
            pltpu.VMEM((128,), jnp.int32),
            pltpu.SemaphoreType.DMA,
        ],
    )
    return fn(dens3, feats)


def _semantic_kernel(feats_ref, qn_ref, cen_ref, tv_ref, out_ref):
    cen = cen_ref[0:_NCLUS, :]
    tv = tv_ref[:, 0:_NCLUS]
    priors = tv / (jnp.sum(tv) + 1e-8)

    ones = jnp.ones((1, cen.shape[1]), jnp.float32)
    cn = jax.lax.dot_general(ones, cen * cen, (((1,), (1,)), ((), ())),
                             preferred_element_type=jnp.float32)
    prod = jax.lax.dot_general(feats_ref[...], cen, (((1,), (1,)), ((), ())),
                               preferred_element_type=jnp.float32)
    d2 = jnp.maximum(qn_ref[...] + cn - 2.0 * prod, 1e-12)
    dist = jnp.sqrt(d2)
    logits = -dist / _TEMP
    logits = logits - jnp.max(logits, axis=1, keepdims=True)
    e = jnp.exp(logits)
    soft = e / jnp.sum(e, axis=1, keepdims=True)
    out_ref[...] = jnp.sum(soft * priors, axis=1, keepdims=True)


def kernel(features, Wc, gamma, beta, memory_bank, perm):
    del perm
    b, c, h, w = features.shape
    n = b * h * w
    m_total = memory_bank.shape[0]
    n_tail = m_total - n

    flat = features.reshape(b, c, h * w).transpose(0, 2, 1).reshape(n, c)
    mtail = memory_bank[n:]

    feats, qn, mn = pl.pallas_call(
        _proj_kernel,
        out_shape=(
            jax.ShapeDtypeStruct((n, c), jnp.float32),
            jax.ShapeDtypeStruct((n, 1), jnp.float32),
            jax.ShapeDtypeStruct((1, m_total), jnp.float32),
        ),
    )(flat, Wc, gamma.reshape(1, c), beta.reshape(1, c), mtail)

    m_all = jnp.concatenate([feats, mtail], axis=0)

    grid = n // _TQ
    dens = pl.pallas_call(
        _density_kernel,
        grid=(grid,),
        in_specs=[
            pl.BlockSpec((_TQ, c), lambda i: (i, 0)),
            pl.BlockSpec((_TQ, 1), lambda i: (i, 0)),
            pl.BlockSpec((m_total, c), lambda i: (0, 0)),
            pl.BlockSpec((1, m_total), lambda i: (0, 0)),
        ],
        out_specs=pl.BlockSpec((_TQ, 1), lambda i: (i, 0)),
        out_shape=jax.ShapeDtypeStruct((n, 1), jnp.float32),
    )(feats, qn, m_all, mn)

    dens3 = dens.reshape(b, h * w)[b - 1]
    cen16, tv16 = _sc_peaks(dens3, feats, (b - 1) * h * w)

    sem = pl.pallas_call(
        _semantic_kernel,
        out_shape=jax.ShapeDtypeStruct((n, 1), jnp.float32),
    )(feats, qn, cen16, tv16.reshape(1, 16))

    return sem.reshape(b, 1, h, w)


# speedup vs baseline: 1.7251x; 1.7142x over previous
import functools

import jax
import jax.numpy as jnp
from jax import lax
from jax.experimental import pallas as pl
from jax.experimental.pallas import tpu as pltpu
from jax.experimental.pallas import tpu_sc as plsc

_K = 64
_NCLUS = 8
_TEMP = 0.1
_BN_EPS = 1e-5
_TQ = 256


def _proj_kernel(f_ref, wc_ref, gamma_ref, beta_ref, mtail_ref,
                 feats_ref, qn_ref, mn_ref):
    x = jax.lax.dot_general(f_ref[...], wc_ref[...], (((1,), (1,)), ((), ())),
                            preferred_element_type=jnp.float32)
    mean = jnp.mean(x, axis=0, keepdims=True)
    var = jnp.mean((x - mean) ** 2, axis=0, keepdims=True)
    xn = (x - mean) / jnp.sqrt(var + _BN_EPS)
    feats = jnp.maximum(xn * gamma_ref[...] + beta_ref[...], 0.0)
    feats_ref[...] = feats

    qn = jnp.sum(feats * feats, axis=1, keepdims=True)
    qn_ref[...] = qn

    ones = jnp.ones((1, feats.shape[1]), jnp.float32)
    mn_q = jax.lax.dot_general(ones, feats * feats, (((1,), (1,)), ((), ())),
                               preferred_element_type=jnp.float32)
    mtail = mtail_ref[...]
    mn_t = jax.lax.dot_general(ones, mtail * mtail, (((1,), (1,)), ((), ())),
                               preferred_element_type=jnp.float32)
    mn_ref[...] = jnp.concatenate([mn_q, mn_t], axis=1)


def _density_kernel(q_ref, qn_ref, m_ref, mn_ref, dens_ref):
    prod = jax.lax.dot_general(q_ref[...], m_ref[...], (((1,), (1,)), ((), ())),
                               preferred_element_type=jnp.float32)
    d2 = jnp.maximum(qn_ref[...] + mn_ref[...] - 2.0 * prod, 1e-12)
    tq, mm = d2.shape

    lo = jax.lax.bitcast_convert_type(jnp.min(d2, axis=1, keepdims=True),
                                      jnp.int32)
    hi = jax.lax.bitcast_convert_type(jnp.max(d2, axis=1, keepdims=True),
                                      jnp.int32)

    def body(_, carry):
        lo, hi = carry
        mid = lo + ((hi - lo) >> 1)
        t = jax.lax.bitcast_convert_type(mid, jnp.float32)
        cnt = jnp.sum(jnp.where(d2 <= t, 1.0, 0.0), axis=1, keepdims=True)
        ge = cnt >= float(_K)
        return jnp.where(ge, lo, mid + 1), jnp.where(ge, mid, hi)

    lo, hi = jax.lax.fori_loop(0, 31, body, (lo, hi))
    r2k = jax.lax.bitcast_convert_type(hi, jnp.float32)

    inv_bw2 = 1.0 / jnp.maximum(r2k, 1e-16)
    dens_ref[...] = jnp.sum(jnp.exp(-d2 * inv_bw2), axis=1, keepdims=True)


def _sc_peaks_body(dens_hbm, feats_hbm, cen_hbm, tv_hbm,
                   dens_v, cen_v, tv_v, idx_v, red_v, redi_v, sem, *, base):
    wid = lax.axis_index("s") * 2 + lax.axis_index("c")

    @pl.when(wid == 0)
    def _():
        pltpu.sync_copy(dens_hbm, dens_v)
        hw = dens_hbm.shape[0]
        nv = hw // 16
        lane = lax.iota(jnp.int32, 16)

        def splat_max_f32(x):
            for s in (8, 4, 2, 1):
                red_v[pl.ds(0, 16)] = x
                x = jnp.maximum(x, plsc.load_gather(red_v, [lane ^ s]))
            return x

        def splat_min_i32(x):
            for s in (8, 4, 2, 1):
                redi_v[pl.ds(0, 16)] = x
                x = jnp.minimum(x, plsc.load_gather(redi_v, [lane ^ s]))
            return x

        def round_body(r, carry):
            tv_vec, idx_vec = carry

            def mx_body(v, best):
                return jnp.maximum(best, dens_v[pl.ds(v * 16, 16)])
            best = lax.fori_loop(0, nv, mx_body,
                                 jnp.full((16,), -jnp.inf, jnp.float32))
            m = splat_max_f32(best)

            def ix_body(v, besti):
                x = dens_v[pl.ds(v * 16, 16)]
                gi = lane + v * 16
                return jnp.minimum(besti, jnp.where(x == m, gi,
                                                    jnp.int32(2 ** 30)))
            besti = lax.fori_loop(0, nv, ix_body,
                                  jnp.full((16,), 2 ** 30, jnp.int32))
            idx = splat_min_i32(besti)

            tv_vec = jnp.where(lane == r, m, tv_vec)
            idx_vec = jnp.where(lane == r, idx, idx_vec)
            plsc.store_scatter(dens_v, [idx],
                               jnp.full((16,), -jnp.inf, jnp.float32),
                               mask=lane == 0)
            return tv_vec, idx_vec

        tv_vec, idx_vec = lax.fori_loop(
            0, _NCLUS, round_body,
            (jnp.zeros((16,), jnp.float32), jnp.zeros((16,), jnp.int32)))

        tv_v[...] = tv_vec
        idx_v[...] = idx_vec + base

        pltpu.async_copy(feats_hbm.at[idx_v], cen_v, sem).wait()
        pltpu.sync_copy(cen_v, cen_hbm)
        pltpu.sync_copy(tv_v, tv_hbm)


def _sc_peaks(dens3, feats, base):
    c = feats.shape[1]
    hw = dens3.shape[0]
    mesh = plsc.VectorSubcoreMesh(core_axis_name="c", subcore_axis_name="s")
    fn = pl.kernel(
        functools.partial(_sc_peaks_body, base=base),
        mesh=mesh,
        compiler_params=pltpu.CompilerParams(needs_layout_passes=False),
        out_type=(
            jax.ShapeDtypeStruct((16, c), jnp.float32),
            jax.ShapeDtypeStruct((16,), jnp.float32),
        ),
        scratch_types=[
            pltpu.VMEM((hw,), jnp.float32),
            pltpu.VMEM((16, c), jnp.float32),
            pltpu.VMEM((16,), jnp.float32),
            pltpu.VMEM((16,), jnp.int32),
            pltpu.VMEM((128,), jnp.float32),
            pltpu.VMEM((128,), jnp.int32),
            pltpu.SemaphoreType.DMA,
        ],
    )
    return fn(dens3, feats)


def _semantic_kernel(feats_ref, qn_ref, cen_ref, tv_ref, out_ref):
    cen = cen_ref[0:_NCLUS, :]
    tv = tv_ref[:, 0:_NCLUS]
    priors = tv / (jnp.sum(tv) + 1e-8)

    ones = jnp.ones((1, cen.shape[1]), jnp.float32)
    cn = jax.lax.dot_general(ones, cen * cen, (((1,), (1,)), ((), ())),
                             preferred_element_type=jnp.float32)
    prod = jax.lax.dot_general(feats_ref[...], cen, (((1,), (1,)), ((), ())),
                               preferred_element_type=jnp.float32)
    d2 = jnp.maximum(qn_ref[...] + cn - 2.0 * prod, 1e-12)
    dist = jnp.sqrt(d2)
    logits = -dist / _TEMP
    logits = logits - jnp.max(logits, axis=1, keepdims=True)
    e = jnp.exp(logits)
    soft = e / jnp.sum(e, axis=1, keepdims=True)
    out_ref[...] = jnp.sum(soft * priors, axis=1, keepdims=True)


def kernel(features, Wc, gamma, beta, memory_bank, perm):
    del perm
    b, c, h, w = features.shape
    n = b * h * w
    m_total = memory_bank.shape[0]
    n_tail = m_total - n

    flat = features.reshape(b, c, h * w).transpose(0, 2, 1).reshape(n, c)
    mtail = memory_bank[n:]

    feats, qn, mn = pl.pallas_call(
        _proj_kernel,
        out_shape=(
            jax.ShapeDtypeStruct((n, c), jnp.float32),
            jax.ShapeDtypeStruct((n, 1), jnp.float32),
            jax.ShapeDtypeStruct((1, m_total), jnp.float32),
        ),
    )(flat, Wc, gamma.reshape(1, c), beta.reshape(1, c), mtail)

    m_all = jnp.concatenate([feats, mtail], axis=0)

    grid = n // _TQ
    dens = pl.pallas_call(
        _density_kernel,
        grid=(grid,),
        in_specs=[
            pl.BlockSpec((_TQ, c), lambda i: (i, 0)),
            pl.BlockSpec((_TQ, 1), lambda i: (i, 0)),
            pl.BlockSpec((m_total, c), lambda i: (0, 0)),
            pl.BlockSpec((1, m_total), lambda i: (0, 0)),
        ],
        out_specs=pl.BlockSpec((_TQ, 1), lambda i: (i, 0)),
        out_shape=jax.ShapeDtypeStruct((n, 1), jnp.float32),
    )(feats, qn, m_all, mn)

    dens3 = dens.reshape(b, h * w)[b - 1]
    cen16, tv16 = _sc_peaks(dens3, feats, (b - 1) * h * w)

    sem = pl.pallas_call(
        _semantic_kernel,
        out_shape=jax.ShapeDtypeStruct((n, 1), jnp.float32),
    )(feats, qn, cen16, tv16.reshape(1, 16))

    return sem.reshape(b, 1, h, w)


# speedup vs baseline: 1.7806x; 1.0322x over previous
import functools

import jax
import jax.numpy as jnp
from jax import lax
from jax.experimental import pallas as pl
from jax.experimental.pallas import tpu as pltpu
from jax.experimental.pallas import tpu_sc as plsc

_K = 64
_NCLUS = 8
_TEMP = 0.1
_BN_EPS = 1e-5
_TQ = 512


def _proj_kernel(f_ref, wc_ref, gamma_ref, beta_ref, mtail_ref,
                 feats_ref, qn_ref, mn_ref):
    x = jax.lax.dot_general(f_ref[...], wc_ref[...], (((1,), (1,)), ((), ())),
                            preferred_element_type=jnp.float32)
    mean = jnp.mean(x, axis=0, keepdims=True)
    var = jnp.mean((x - mean) ** 2, axis=0, keepdims=True)
    xn = (x - mean) / jnp.sqrt(var + _BN_EPS)
    feats = jnp.maximum(xn * gamma_ref[...] + beta_ref[...], 0.0)
    feats_ref[...] = feats

    qn = jnp.sum(feats * feats, axis=1, keepdims=True)
    qn_ref[...] = qn

    ones = jnp.ones((1, feats.shape[1]), jnp.float32)
    mn_q = jax.lax.dot_general(ones, feats * feats, (((1,), (1,)), ((), ())),
                               preferred_element_type=jnp.float32)
    mtail = mtail_ref[...]
    mn_t = jax.lax.dot_general(ones, mtail * mtail, (((1,), (1,)), ((), ())),
                               preferred_element_type=jnp.float32)
    mn_ref[...] = jnp.concatenate([mn_q, mn_t], axis=1)


def _density_kernel(q_ref, qn_ref, m_ref, mn_ref, dens_ref):
    prod = jax.lax.dot_general(q_ref[...], m_ref[...], (((1,), (1,)), ((), ())),
                               preferred_element_type=jnp.float32)
    d2 = jnp.maximum(qn_ref[...] + mn_ref[...] - 2.0 * prod, 1e-12)
    tq, mm = d2.shape

    lo = jax.lax.bitcast_convert_type(jnp.min(d2, axis=1, keepdims=True),
                                      jnp.int32)
    hi = jax.lax.bitcast_convert_type(jnp.max(d2, axis=1, keepdims=True),
                                      jnp.int32)

    def body(_, carry):
        lo, hi = carry
        mid = lo + ((hi - lo) >> 1)
        t = jax.lax.bitcast_convert_type(mid, jnp.float32)
        cnt = jnp.sum(jnp.where(d2 <= t, 1.0, 0.0), axis=1, keepdims=True)
        ge = cnt >= float(_K)
        return jnp.where(ge, lo, mid + 1), jnp.where(ge, mid, hi)

    lo, hi = jax.lax.fori_loop(0, 31, body, (lo, hi))
    r2k = jax.lax.bitcast_convert_type(hi, jnp.float32)

    inv_bw2 = 1.0 / jnp.maximum(r2k, 1e-16)
    dens_ref[...] = jnp.sum(jnp.exp(-d2 * inv_bw2), axis=1, keepdims=True)


def _sc_peaks_body(dens_hbm, feats_hbm, cen_hbm, tv_hbm,
                   dens_v, cen_v, tv_v, idx_v, red_v, redi_v, sem, *, base):
    wid = lax.axis_index("s") * 2 + lax.axis_index("c")

    @pl.when(wid == 0)
    def _():
        pltpu.sync_copy(dens_hbm, dens_v)
        hw = dens_hbm.shape[0]
        nv = hw // 16
        lane = lax.iota(jnp.int32, 16)

        def splat_max_f32(x):
            for s in (8, 4, 2, 1):
                red_v[pl.ds(0, 16)] = x
                x = jnp.maximum(x, plsc.load_gather(red_v, [lane ^ s]))
            return x

        def splat_min_i32(x):
            for s in (8, 4, 2, 1):
                redi_v[pl.ds(0, 16)] = x
                x = jnp.minimum(x, plsc.load_gather(redi_v, [lane ^ s]))
            return x

        def round_body(r, carry):
            tv_vec, idx_vec = carry

            def mx_body(v, best):
                return jnp.maximum(best, dens_v[pl.ds(v * 16, 16)])
            best = lax.fori_loop(0, nv, mx_body,
                                 jnp.full((16,), -jnp.inf, jnp.float32))
            m = splat_max_f32(best)

            def ix_body(v, besti):
                x = dens_v[pl.ds(v * 16, 16)]
                gi = lane + v * 16
                return jnp.minimum(besti, jnp.where(x == m, gi,
                                                    jnp.int32(2 ** 30)))
            besti = lax.fori_loop(0, nv, ix_body,
                                  jnp.full((16,), 2 ** 30, jnp.int32))
            idx = splat_min_i32(besti)

            tv_vec = jnp.where(lane == r, m, tv_vec)
            idx_vec = jnp.where(lane == r, idx, idx_vec)
            plsc.store_scatter(dens_v, [idx],
                               jnp.full((16,), -jnp.inf, jnp.float32),
                               mask=lane == 0)
            return tv_vec, idx_vec

        tv_vec, idx_vec = lax.fori_loop(
            0, _NCLUS, round_body,
            (jnp.zeros((16,), jnp.float32), jnp.zeros((16,), jnp.int32)))

        tv_v[...] = tv_vec
        idx_v[...] = idx_vec + base

        pltpu.async_copy(feats_hbm.at[idx_v], cen_v, sem).wait()
        pltpu.sync_copy(cen_v, cen_hbm)
        pltpu.sync_copy(tv_v, tv_hbm)


def _sc_peaks(dens3, feats, base):
    c = feats.shape[1]
    hw = dens3.shape[0]
    mesh = plsc.VectorSubcoreMesh(core_axis_name="c", subcore_axis_name="s")
    fn = pl.kernel(
        functools.partial(_sc_peaks_body, base=base),
        mesh=mesh,
        compiler_params=pltpu.CompilerParams(needs_layout_passes=False),
        out_type=(
            jax.ShapeDtypeStruct((16, c), jnp.float32),
            jax.ShapeDtypeStruct((16,), jnp.float32),
        ),
        scratch_types=[
            pltpu.VMEM((hw,), jnp.float32),
            pltpu.VMEM((16, c), jnp.float32),
            pltpu.VMEM((16,), jnp.float32),
            pltpu.VMEM((16,), jnp.int32),
            pltpu.VMEM((128,), jnp.float32),
            pltpu.VMEM((128,), jnp.int32),
            pltpu.SemaphoreType.DMA,
        ],
    )
    return fn(dens3, feats)


def _semantic_kernel(feats_ref, qn_ref, cen_ref, tv_ref, out_ref):
    cen = cen_ref[0:_NCLUS, :]
    tv = tv_ref[:, 0:_NCLUS]
    priors = tv / (jnp.sum(tv) + 1e-8)

    ones = jnp.ones((1, cen.shape[1]), jnp.float32)
    cn = jax.lax.dot_general(ones, cen * cen, (((1,), (1,)), ((), ())),
                             preferred_element_type=jnp.float32)
    prod = jax.lax.dot_general(feats_ref[...], cen, (((1,), (1,)), ((), ())),
                               preferred_element_type=jnp.float32)
    d2 = jnp.maximum(qn_ref[...] + cn - 2.0 * prod, 1e-12)
    dist = jnp.sqrt(d2)
    logits = -dist / _TEMP
    logits = logits - jnp.max(logits, axis=1, keepdims=True)
    e = jnp.exp(logits)
    soft = e / jnp.sum(e, axis=1, keepdims=True)
    out_ref[...] = jnp.sum(soft * priors, axis=1, keepdims=True)


def kernel(features, Wc, gamma, beta, memory_bank, perm):
    del perm
    b, c, h, w = features.shape
    n = b * h * w
    m_total = memory_bank.shape[0]
    n_tail = m_total - n

    flat = features.reshape(b, c, h * w).transpose(0, 2, 1).reshape(n, c)
    mtail = memory_bank[n:]

    feats, qn, mn = pl.pallas_call(
        _proj_kernel,
        out_shape=(
            jax.ShapeDtypeStruct((n, c), jnp.float32),
            jax.ShapeDtypeStruct((n, 1), jnp.float32),
            jax.ShapeDtypeStruct((1, m_total), jnp.float32),
        ),
    )(flat, Wc, gamma.reshape(1, c), beta.reshape(1, c), mtail)

    m_all = jnp.concatenate([feats, mtail], axis=0)

    grid = n // _TQ
    dens = pl.pallas_call(
        _density_kernel,
        grid=(grid,),
        in_specs=[
            pl.BlockSpec((_TQ, c), lambda i: (i, 0)),
            pl.BlockSpec((_TQ, 1), lambda i: (i, 0)),
            pl.BlockSpec((m_total, c), lambda i: (0, 0)),
            pl.BlockSpec((1, m_total), lambda i: (0, 0)),
        ],
        out_specs=pl.BlockSpec((_TQ, 1), lambda i: (i, 0)),
        out_shape=jax.ShapeDtypeStruct((n, 1), jnp.float32),
        compiler_params=pltpu.CompilerParams(
            vmem_limit_bytes=60000 * 1024),
    )(feats, qn, m_all, mn)

    dens3 = dens.reshape(b, h * w)[b - 1]
    cen16, tv16 = _sc_peaks(dens3, feats, (b - 1) * h * w)

    sem = pl.pallas_call(
        _semantic_kernel,
        out_shape=jax.ShapeDtypeStruct((n, 1), jnp.float32),
    )(feats, qn, cen16, tv16.reshape(1, 16))

    return sem.reshape(b, 1, h, w)
